# baseline (speedup 1.0000x reference)
import functools

import jax
import jax.numpy as jnp
from jax import lax
from jax.experimental import pallas as pl
from jax.experimental.pallas import tpu as pltpu
from jax.experimental.pallas import tpu_sc as plsc

_B = 4096
_L = 200
_DIM = 16
_N = _B * _L
_D = 2 * _DIM

_NC = 2
_NS = 16
_NW = _NC * _NS
_PER_W = _N // _NW

_SUB = 128
_KSUB = 8
_CHUNK = _SUB * _KSUB
_STEPS = _PER_W // _CHUNK
_ROW_BLOCKS_PER_W = _PER_W // _SUB


def _build():
    mesh = plsc.VectorSubcoreMesh(core_axis_name="c", subcore_axis_name="s")

    @functools.partial(
        pl.kernel,
        mesh=mesh,
        out_type=jax.ShapeDtypeStruct((_N, _D), jnp.float32),
        compiler_params=pltpu.CompilerParams(use_tc_tiling_on_sc=False),
        scratch_types=[
            pltpu.VMEM((_KSUB, _SUB), jnp.int32),
            pltpu.VMEM((_CHUNK, _D), jnp.float32),
            pltpu.SemaphoreType.DMA,
            pltpu.SemaphoreType.DMA,
        ],
    )
    def gather_rows(ids_hbm, table_hbm, out_hbm, idx_v, rows_v, gsem, osem):
        wid = lax.axis_index("s") * _NC + lax.axis_index("c")
        row_base = wid * _ROW_BLOCKS_PER_W
        out_base = wid * _PER_W

        def step(i, carry):
            pltpu.sync_copy(ids_hbm.at[pl.ds(row_base + i * _KSUB, _KSUB)], idx_v)

            copies = []
            for j in range(_KSUB):
                copies.append(
                    pltpu.async_copy(
                        table_hbm.at[idx_v.at[j]],
                        rows_v.at[pl.ds(j * _SUB, _SUB)],
                        gsem,
                    )
                )
            for c in copies:
                c.wait()

            pltpu.async_copy(
                rows_v, out_hbm.at[pl.ds(out_base + i * _CHUNK, _CHUNK)], osem
            ).wait()
            return carry

        lax.fori_loop(0, _STEPS, step, 0)

    return gather_rows


_GATHER = _build()


def kernel(ids, embed_weight):
    ids2d = ids.reshape(_N // _SUB, _SUB)
    flat = _GATHER(ids2d, embed_weight)
    emb = flat.reshape(_B, _L, _D)
    return lax.complex(emb[..., :_DIM], emb[..., _DIM:])


# speedup vs baseline: 1.0283x; 1.0283x over previous
import functools

import jax
import jax.numpy as jnp
from jax import lax
from jax.experimental import pallas as pl
from jax.experimental.pallas import tpu as pltpu
from jax.experimental.pallas import tpu_sc as plsc

_B = 4096
_L = 200
_DIM = 16
_N = _B * _L
_D = 2 * _DIM

_NC = 2
_NS = 16
_NW = _NC * _NS
_PER_W = _N // _NW

_SUB = 128
_KSUB = 8
_CHUNK = _SUB * _KSUB
_STEPS = _PER_W // _CHUNK
_ROW_BLOCKS_PER_W = _PER_W // _SUB


def _build():
    mesh = plsc.VectorSubcoreMesh(core_axis_name="c", subcore_axis_name="s")

    @functools.partial(
        pl.kernel,
        mesh=mesh,
        out_type=(
            jax.ShapeDtypeStruct((_N, _DIM), jnp.float32),
            jax.ShapeDtypeStruct((_N, _DIM), jnp.float32),
        ),
        compiler_params=pltpu.CompilerParams(use_tc_tiling_on_sc=False),
        scratch_types=[
            pltpu.VMEM((_KSUB, _SUB), jnp.int32),
            pltpu.VMEM((_CHUNK, _D), jnp.float32),
            pltpu.SemaphoreType.DMA,
            pltpu.SemaphoreType.DMA,
        ],
    )
    def gather_rows(ids_hbm, table_hbm, re_hbm, im_hbm, idx_v, rows_v, gsem, osem):
        wid = lax.axis_index("s") * _NC + lax.axis_index("c")
        row_base = wid * _ROW_BLOCKS_PER_W
        out_base = wid * _PER_W

        def step(i, carry):
            pltpu.sync_copy(ids_hbm.at[pl.ds(row_base + i * _KSUB, _KSUB)], idx_v)

            copies = []
            for j in range(_KSUB):
                copies.append(
                    pltpu.async_copy(
                        table_hbm.at[idx_v.at[j]],
                        rows_v.at[pl.ds(j * _SUB, _SUB)],
                        gsem,
                    )
                )
            for c in copies:
                c.wait()

            dst = pl.ds(out_base + i * _CHUNK, _CHUNK)
            c_re = pltpu.async_copy(
                rows_v.at[:, pl.ds(0, _DIM)], re_hbm.at[dst], osem
            )
            c_im = pltpu.async_copy(
                rows_v.at[:, pl.ds(_DIM, _DIM)], im_hbm.at[dst], osem
            )
            c_re.wait()
            c_im.wait()
            return carry

        lax.fori_loop(0, _STEPS, step, 0)

    return gather_rows


_GATHER = _build()


def kernel(ids, embed_weight):
    ids2d = ids.reshape(_N // _SUB, _SUB)
    re, im = _GATHER(ids2d, embed_weight)
    return lax.complex(re.reshape(_B, _L, _DIM), im.reshape(_B, _L, _DIM))


# speedup vs baseline: 1.1076x; 1.0771x over previous
import functools

import jax
import jax.numpy as jnp
from jax import lax
from jax.experimental import pallas as pl
from jax.experimental.pallas import tpu as pltpu
from jax.experimental.pallas import tpu_sc as plsc

_B = 4096
_L = 200
_DIM = 16
_N = _B * _L
_D = 2 * _DIM

_NC = 2
_NS = 16
_NW = _NC * _NS
_PER_W = _N // _NW

_SUB = 128
_KSUB = 8
_CHUNK = _SUB * _KSUB
_STEPS = _PER_W // _CHUNK
_ROW_BLOCKS_PER_W = _PER_W // _SUB


def _build():
    mesh = plsc.VectorSubcoreMesh(core_axis_name="c", subcore_axis_name="s")

    @functools.partial(
        pl.kernel,
        mesh=mesh,
        out_type=(
            jax.ShapeDtypeStruct((_N, _DIM), jnp.float32),
            jax.ShapeDtypeStruct((_N, _DIM), jnp.float32),
        ),
        compiler_params=pltpu.CompilerParams(use_tc_tiling_on_sc=False),
        scratch_types=[
            pltpu.VMEM((_KSUB, _SUB), jnp.int32),
            pltpu.VMEM((_KSUB, _SUB), jnp.int32),
            pltpu.VMEM((_CHUNK, _DIM), jnp.float32),
            pltpu.VMEM((_CHUNK, _DIM), jnp.float32),
            pltpu.SemaphoreType.DMA,
            pltpu.SemaphoreType.DMA,
        ],
    )
    def gather_halves(
        idx_re_hbm, idx_im_hbm, half_hbm, re_hbm, im_hbm,
        idxr_v, idxi_v, re_v, im_v, gsem, osem,
    ):
        wid = lax.axis_index("s") * _NC + lax.axis_index("c")
        row_base = wid * _ROW_BLOCKS_PER_W
        out_base = wid * _PER_W

        def step(i, carry):
            rb = pl.ds(row_base + i * _KSUB, _KSUB)
            pltpu.sync_copy(idx_re_hbm.at[rb], idxr_v)
            pltpu.sync_copy(idx_im_hbm.at[rb], idxi_v)

            copies = []
            for j in range(_KSUB):
                dst = pl.ds(j * _SUB, _SUB)
                copies.append(
                    pltpu.async_copy(half_hbm.at[idxr_v.at[j]], re_v.at[dst], gsem)
                )
                copies.append(
                    pltpu.async_copy(half_hbm.at[idxi_v.at[j]], im_v.at[dst], gsem)
                )
            for c in copies:
                c.wait()

            dst = pl.ds(out_base + i * _CHUNK, _CHUNK)
            c_re = pltpu.async_copy(re_v, re_hbm.at[dst], osem)
            c_im = pltpu.async_copy(im_v, im_hbm.at[dst], osem)
            c_re.wait()
            c_im.wait()
            return carry

        lax.fori_loop(0, _STEPS, step, 0)

    return gather_halves


_GATHER = _build()


def kernel(ids, embed_weight):
    flat_ids = ids.reshape(_N // _SUB, _SUB)
    idx_re = flat_ids * 2
    idx_im = idx_re + 1
    halves = embed_weight.reshape(2 * 1000000, _DIM)
    re, im = _GATHER(idx_re, idx_im, halves)
    return lax.complex(re.reshape(_B, _L, _DIM), im.reshape(_B, _L, _DIM))


# speedup vs baseline: 1.3952x; 1.2597x over previous
import functools

import jax
import jax.numpy as jnp
from jax import lax
from jax.experimental import pallas as pl
from jax.experimental.pallas import tpu as pltpu
from jax.experimental.pallas import tpu_sc as plsc

_B = 4096
_L = 200
_DIM = 16
_N = _B * _L
_V = 1000000

_NC = 2
_NS = 16
_NW = _NC * _NS
_BB = _B // _NW


def _build():
    mesh = plsc.VectorSubcoreMesh(core_axis_name="c", subcore_axis_name="s")

    @functools.partial(
        pl.kernel,
        mesh=mesh,
        out_type=(
            jax.ShapeDtypeStruct((_L, 2, _NW, 8, _BB), jnp.float32),
            jax.ShapeDtypeStruct((_L, 2, _NW, 8, _BB), jnp.float32),
        ),
        compiler_params=pltpu.CompilerParams(
            use_tc_tiling_on_sc=False, needs_layout_passes=False
        ),
        scratch_types=[
            pltpu.VMEM((_L, _BB), jnp.int32),
            pltpu.VMEM((_L, _BB), jnp.int32),
            pltpu.VMEM((_BB, _DIM), jnp.float32),
            pltpu.VMEM((_BB, _DIM), jnp.float32),
            pltpu.VMEM((2, 8, _BB), jnp.float32),
            pltpu.VMEM((2, 8, _BB), jnp.float32),
            pltpu.SemaphoreType.DMA,
            pltpu.SemaphoreType.DMA,
        ],
    )
    def gather_planes(
        idx_re_hbm, idx_im_hbm, half_hbm, re_hbm, im_hbm,
        idxr_v, idxi_v, re_g, im_g, re_t, im_t, gsem, osem,
    ):
        wid = lax.axis_index("s") * _NC + lax.axis_index("c")

        rows = pl.ds(wid * _L, _L)
        pltpu.sync_copy(idx_re_hbm.at[rows], idxr_v)
        pltpu.sync_copy(idx_im_hbm.at[rows], idxi_v)

        lane = lax.iota(jnp.int32, 16)

        def step(l, carry):
            c_re = pltpu.async_copy(half_hbm.at[idxr_v.at[l]], re_g, gsem)
            c_im = pltpu.async_copy(half_hbm.at[idxi_v.at[l]], im_g, gsem)
            c_re.wait()
            c_im.wait()

            for k in range(_DIM):
                t1 = jnp.full((16,), k // 8, dtype=jnp.int32)
                e0 = jnp.full((16,), k % 8, dtype=jnp.int32)
                kv = jnp.full((16,), k, dtype=jnp.int32)
                for m in range(_BB // 16):
                    jv = m * 16 + lane
                    plsc.store_scatter(
                        re_t, [t1, e0, jv], plsc.load_gather(re_g, [jv, kv])
                    )
                    plsc.store_scatter(
                        im_t, [t1, e0, jv], plsc.load_gather(im_g, [jv, kv])
                    )

            c_or = pltpu.async_copy(re_t, re_hbm.at[l, :, wid], osem)
            c_oi = pltpu.async_copy(im_t, im_hbm.at[l, :, wid], osem)
            c_or.wait()
            c_oi.wait()
            return carry

        lax.fori_loop(0, _L, step, 0)

    return gather_planes


_GATHER = _build()


def kernel(ids, embed_weight):
    blocked = ids.reshape(_NW, _BB, _L).transpose(0, 2, 1)
    idx_re = (blocked * 2).reshape(_NW * _L, _BB)
    idx_im = idx_re + 1
    halves = embed_weight.reshape(2 * _V, _DIM)
    re4, im4 = _GATHER(idx_re, idx_im, halves)

    def conv(x):
        return x.transpose(2, 4, 0, 1, 3).reshape(_B, _L, _DIM)

    return lax.complex(conv(re4), conv(im4))


# speedup vs baseline: 1.4543x; 1.0423x over previous
import functools

import jax
import jax.numpy as jnp
from jax import lax
from jax.experimental import pallas as pl
from jax.experimental.pallas import tpu as pltpu
from jax.experimental.pallas import tpu_sc as plsc

_B = 4096
_L = 200
_DIM = 16
_N = _B * _L
_V = 1000000

_NC = 2
_NS = 16
_NW = _NC * _NS
_BB = _B // _NW


def _build():
    mesh = plsc.VectorSubcoreMesh(core_axis_name="c", subcore_axis_name="s")

    @functools.partial(
        pl.kernel,
        mesh=mesh,
        out_type=(
            jax.ShapeDtypeStruct((_L, 2, _NW, 8, _BB), jnp.float32),
            jax.ShapeDtypeStruct((_L, 2, _NW, 8, _BB), jnp.float32),
        ),
        compiler_params=pltpu.CompilerParams(
            use_tc_tiling_on_sc=False, needs_layout_passes=False
        ),
        scratch_types=[
            pltpu.VMEM((_L, _BB), jnp.int32),
            pltpu.VMEM((_L, _BB), jnp.int32),
            pltpu.VMEM((_BB, _DIM), jnp.float32),
            pltpu.VMEM((_BB, _DIM), jnp.float32),
            pltpu.VMEM((_BB, _DIM), jnp.float32),
            pltpu.VMEM((_BB, _DIM), jnp.float32),
            pltpu.VMEM((2, 8, _BB), jnp.float32),
            pltpu.VMEM((2, 8, _BB), jnp.float32),
            pltpu.VMEM((2, 8, _BB), jnp.float32),
            pltpu.VMEM((2, 8, _BB), jnp.float32),
            pltpu.SemaphoreType.DMA,
            pltpu.SemaphoreType.DMA,
            pltpu.SemaphoreType.DMA,
            pltpu.SemaphoreType.DMA,
        ],
    )
    def gather_planes(
        idx_re_hbm, idx_im_hbm, half_hbm, re_hbm, im_hbm,
        idxr_v, idxi_v, rg0, ig0, rg1, ig1, rt0, it0, rt1, it1,
        gsem0, gsem1, osem0, osem1,
    ):
        wid = lax.axis_index("s") * _NC + lax.axis_index("c")

        rows = pl.ds(wid * _L, _L)
        pltpu.sync_copy(idx_re_hbm.at[rows], idxr_v)
        pltpu.sync_copy(idx_im_hbm.at[rows], idxi_v)

        lane = lax.iota(jnp.int32, 16)
        t1l = lane // 8
        e0l = lane % 8

        def fire_gathers(l, rg, ig, gsem):
            pltpu.async_copy(half_hbm.at[idxr_v.at[l]], rg, gsem)
            pltpu.async_copy(half_hbm.at[idxi_v.at[l]], ig, gsem)

        def wait_gathers(l, rg, ig, gsem):
            pltpu.make_async_copy(half_hbm.at[idxr_v.at[l]], rg, gsem).wait()
            pltpu.make_async_copy(half_hbm.at[idxi_v.at[l]], ig, gsem).wait()

        def transpose(src, dst):
            for j in range(_BB):
                jv = jnp.full((16,), j, dtype=jnp.int32)
                plsc.store_scatter(dst, [t1l, e0l, jv], src[j, :])

        def fire_out(l, rt, it, osem):
            pltpu.async_copy(rt, re_hbm.at[l, :, wid], osem)
            pltpu.async_copy(it, im_hbm.at[l, :, wid], osem)

        def wait_out(l, rt, it, osem):
            pltpu.make_async_copy(rt, re_hbm.at[l, :, wid], osem).wait()
            pltpu.make_async_copy(it, im_hbm.at[l, :, wid], osem).wait()

        fire_gathers(0, rg0, ig0, gsem0)

        def body(i, carry):
            l0 = 2 * i
            l1 = l0 + 1
            fire_gathers(l1, rg1, ig1, gsem1)

            wait_gathers(l0, rg0, ig0, gsem0)

            @pl.when(i > 0)
            def _():
                wait_out(l0 - 2, rt0, it0, osem0)

            transpose(rg0, rt0)
            transpose(ig0, it0)
            fire_out(l0, rt0, it0, osem0)

            @pl.when(i < _L // 2 - 1)
            def _():
                fire_gathers(l0 + 2, rg0, ig0, gsem0)

            wait_gathers(l1, rg1, ig1, gsem1)

            @pl.when(i > 0)
            def _():
                wait_out(l1 - 2, rt1, it1, osem1)

            transpose(rg1, rt1)
            transpose(ig1, it1)
            fire_out(l1, rt1, it1, osem1)
            return carry

        lax.fori_loop(0, _L // 2, body, 0)

        wait_out(_L - 2, rt0, it0, osem0)
        wait_out(_L - 1, rt1, it1, osem1)

    return gather_planes


_GATHER = _build()


def kernel(ids, embed_weight):
    blocked = ids.reshape(_NW, _BB, _L).transpose(0, 2, 1)
    idx_re = (blocked * 2).reshape(_NW * _L, _BB)
    idx_im = idx_re + 1
    halves = embed_weight.reshape(2 * _V, _DIM)
    re4, im4 = _GATHER(idx_re, idx_im, halves)

    def conv(x):
        return x.transpose(2, 4, 0, 1, 3).reshape(_B, _L, _DIM)

    return lax.complex(conv(re4), conv(im4))


# speedup vs baseline: 1.5683x; 1.0784x over previous
import functools

import jax
import jax.numpy as jnp
from jax import lax
from jax.experimental import pallas as pl
from jax.experimental.pallas import tpu as pltpu
from jax.experimental.pallas import tpu_sc as plsc

_B = 4096
_L = 200
_DIM = 16
_N = _B * _L
_V = 1000000

_NC = 2
_NS = 16
_NW = _NC * _NS
_BB = _B // _NW


def _build():
    mesh = plsc.VectorSubcoreMesh(core_axis_name="c", subcore_axis_name="s")

    @functools.partial(
        pl.kernel,
        mesh=mesh,
        out_type=(
            jax.ShapeDtypeStruct((_L, 2, _NW, 8, _BB), jnp.float32),
            jax.ShapeDtypeStruct((_L, 2, _NW, 8, _BB), jnp.float32),
        ),
        compiler_params=pltpu.CompilerParams(
            use_tc_tiling_on_sc=False, needs_layout_passes=False
        ),
        scratch_types=[
            pltpu.VMEM((_L, _BB), jnp.int32),
            pltpu.VMEM((_L, _BB), jnp.int32),
            pltpu.VMEM((_BB, _DIM), jnp.float32),
            pltpu.VMEM((_BB, _DIM), jnp.float32),
            pltpu.VMEM((_BB, _DIM), jnp.float32),
            pltpu.VMEM((_BB, _DIM), jnp.float32),
            pltpu.VMEM((2, 8, _BB), jnp.float32),
            pltpu.VMEM((2, 8, _BB), jnp.float32),
            pltpu.VMEM((2, 8, _BB), jnp.float32),
            pltpu.VMEM((2, 8, _BB), jnp.float32),
            pltpu.SemaphoreType.DMA,
            pltpu.SemaphoreType.DMA,
            pltpu.SemaphoreType.DMA,
            pltpu.SemaphoreType.DMA,
        ],
    )
    def gather_planes(
        idx_re_hbm, idx_im_hbm, half_hbm, re_hbm, im_hbm,
        idxr_v, idxi_v, rg0, ig0, rg1, ig1, rt0, it0, rt1, it1,
        gsem0, gsem1, osem0, osem1,
    ):
        wid = lax.axis_index("s") * _NC + lax.axis_index("c")

        rows = pl.ds(wid * _L, _L)
        pltpu.sync_copy(idx_re_hbm.at[rows], idxr_v)
        pltpu.sync_copy(idx_im_hbm.at[rows], idxi_v)

        lane = lax.iota(jnp.int32, 16)
        t1l = lane // 8
        e0l = lane % 8
        rots = [(lane + c) % 16 for c in range(16)]

        def fire_gathers(l, rg, ig, gsem):
            pltpu.async_copy(half_hbm.at[idxr_v.at[l]], rg, gsem)
            pltpu.async_copy(half_hbm.at[idxi_v.at[l]], ig, gsem)

        def wait_gathers(l, rg, ig, gsem):
            pltpu.make_async_copy(half_hbm.at[idxr_v.at[l]], rg, gsem).wait()
            pltpu.make_async_copy(half_hbm.at[idxi_v.at[l]], ig, gsem).wait()

        def transpose(src, dst):
            for c in range(16):
                rot = rots[c]
                for m in range(_BB // 16):
                    jv = m * 16 + rot
                    v = plsc.load_gather(src, [jv, lane])
                    plsc.store_scatter(dst, [t1l, e0l, jv], v)

        def fire_out(l, rt, it, osem):
            pltpu.async_copy(rt, re_hbm.at[l, :, wid], osem)
            pltpu.async_copy(it, im_hbm.at[l, :, wid], osem)

        def wait_out(l, rt, it, osem):
            pltpu.make_async_copy(rt, re_hbm.at[l, :, wid], osem).wait()
            pltpu.make_async_copy(it, im_hbm.at[l, :, wid], osem).wait()

        fire_gathers(0, rg0, ig0, gsem0)

        def body(i, carry):
            l0 = 2 * i
            l1 = l0 + 1
            fire_gathers(l1, rg1, ig1, gsem1)

            wait_gathers(l0, rg0, ig0, gsem0)

            @pl.when(i > 0)
            def _():
                wait_out(l0 - 2, rt0, it0, osem0)

            transpose(rg0, rt0)
            transpose(ig0, it0)
            fire_out(l0, rt0, it0, osem0)

            @pl.when(i < _L // 2 - 1)
            def _():
                fire_gathers(l0 + 2, rg0, ig0, gsem0)

            wait_gathers(l1, rg1, ig1, gsem1)

            @pl.when(i > 0)
            def _():
                wait_out(l1 - 2, rt1, it1, osem1)

            transpose(rg1, rt1)
            transpose(ig1, it1)
            fire_out(l1, rt1, it1, osem1)
            return carry

        lax.fori_loop(0, _L // 2, body, 0)

        wait_out(_L - 2, rt0, it0, osem0)
        wait_out(_L - 1, rt1, it1, osem1)

    return gather_planes


_GATHER = _build()


def kernel(ids, embed_weight):
    blocked = ids.reshape(_NW, _BB, _L).transpose(0, 2, 1)
    idx_re = (blocked * 2).reshape(_NW * _L, _BB)
    idx_im = idx_re + 1
    halves = embed_weight.reshape(2 * _V, _DIM)
    re4, im4 = _GATHER(idx_re, idx_im, halves)

    def conv(x):
        return x.transpose(2, 4, 0, 1, 3).reshape(_B, _L, _DIM)

    return lax.complex(conv(re4), conv(im4))


# speedup vs baseline: 1.6777x; 1.0697x over previous
import functools

import jax
import jax.numpy as jnp
from jax import lax
from jax.experimental import pallas as pl
from jax.experimental.pallas import tpu as pltpu
from jax.experimental.pallas import tpu_sc as plsc

_B = 4096
_L = 200
_DIM = 16
_N = _B * _L
_V = 1000000

_NC = 2
_NS = 16
_NW = _NC * _NS
_BB = _B // _NW
_QROWS = _V // 4


def _build():
    mesh = plsc.VectorSubcoreMesh(core_axis_name="c", subcore_axis_name="s")

    @functools.partial(
        pl.kernel,
        mesh=mesh,
        out_type=(
            jax.ShapeDtypeStruct((_L, 2, _NW, 8, _BB), jnp.float32),
            jax.ShapeDtypeStruct((_L, 2, _NW, 8, _BB), jnp.float32),
        ),
        compiler_params=pltpu.CompilerParams(
            use_tc_tiling_on_sc=True, needs_layout_passes=False
        ),
        scratch_types=[
            pltpu.VMEM((_L, _BB), jnp.int32),
            pltpu.VMEM((_L, _BB), jnp.int32),
            pltpu.VMEM((_BB, _BB), jnp.float32),
            pltpu.VMEM((_BB, _BB), jnp.float32),
            pltpu.VMEM((2, 8, _BB), jnp.float32),
            pltpu.VMEM((2, 8, _BB), jnp.float32),
            pltpu.VMEM((2, 8, _BB), jnp.float32),
            pltpu.VMEM((2, 8, _BB), jnp.float32),
            pltpu.SemaphoreType.DMA,
            pltpu.SemaphoreType.DMA,
            pltpu.SemaphoreType.DMA,
            pltpu.SemaphoreType.DMA,
        ],
    )
    def gather_planes(
        qidx_hbm, off_hbm, quad_hbm, re_hbm, im_hbm,
        qidx_v, off_v, g0, g1, rt0, it0, rt1, it1,
        gsem0, gsem1, osem0, osem1,
    ):
        wid = lax.axis_index("s") * _NC + lax.axis_index("c")

        rows = pl.ds(wid * _L, _L)
        pltpu.sync_copy(qidx_hbm.at[rows], qidx_v)
        pltpu.sync_copy(off_hbm.at[rows], off_v)

        lane = lax.iota(jnp.int32, 16)
        jvs = [m * 16 + lane for m in range(_BB // 16)]

        def fire_gather(l, g, gsem):
            pltpu.async_copy(quad_hbm.at[qidx_v.at[l]], g, gsem)

        def wait_gather(l, g, gsem):
            pltpu.make_async_copy(quad_hbm.at[qidx_v.at[l]], g, gsem).wait()

        def transpose(l, src, dre, dim):
            lv = jnp.full((16,), l, jnp.int32)
            offs = [
                plsc.load_gather(off_v, [lv, jvs[m]])
                for m in range(_BB // 16)
            ]

            def cbody(c, carry):
                rot = (lane + c) % 16
                t1 = rot // 8
                e0 = rot % 8
                for m in range(_BB // 16):
                    jv = jvs[m]
                    col = offs[m] + rot
                    plsc.store_scatter(
                        dre, [t1, e0, jv], plsc.load_gather(src, [jv, col])
                    )
                    plsc.store_scatter(
                        dim, [t1, e0, jv],
                        plsc.load_gather(src, [jv, col + _DIM]),
                    )
                return carry

            lax.fori_loop(0, _DIM, cbody, 0)

        def fire_out(l, rt, it, osem):
            pltpu.async_copy(rt, re_hbm.at[l, :, wid], osem)
            pltpu.async_copy(it, im_hbm.at[l, :, wid], osem)

        def wait_out(l, rt, it, osem):
            pltpu.make_async_copy(rt, re_hbm.at[l, :, wid], osem).wait()
            pltpu.make_async_copy(it, im_hbm.at[l, :, wid], osem).wait()

        fire_gather(0, g0, gsem0)

        def body(i, carry):
            l0 = 2 * i
            l1 = l0 + 1
            fire_gather(l1, g1, gsem1)

            wait_gather(l0, g0, gsem0)

            @pl.when(i > 0)
            def _():
                wait_out(l0 - 2, rt0, it0, osem0)

            transpose(l0, g0, rt0, it0)
            fire_out(l0, rt0, it0, osem0)

            @pl.when(i < _L // 2 - 1)
            def _():
                fire_gather(l0 + 2, g0, gsem0)

            wait_gather(l1, g1, gsem1)

            @pl.when(i > 0)
            def _():
                wait_out(l1 - 2, rt1, it1, osem1)

            transpose(l1, g1, rt1, it1)
            fire_out(l1, rt1, it1, osem1)
            return carry

        lax.fori_loop(0, _L // 2, body, 0)

        wait_out(_L - 2, rt0, it0, osem0)
        wait_out(_L - 1, rt1, it1, osem1)

    return gather_planes


_GATHER = _build()


def kernel(ids, embed_weight):
    blocked = ids.reshape(_NW, _BB, _L).transpose(0, 2, 1)
    qidx = (blocked // 4).reshape(_NW * _L, _BB)
    off = ((blocked % 4) * 32).reshape(_NW * _L, _BB)
    quads = embed_weight.reshape(_QROWS, 128)
    re4, im4 = _GATHER(qidx, off, quads)

    def conv(x):
        return x.transpose(2, 4, 0, 1, 3).reshape(_B, _L, _DIM)

    return lax.complex(conv(re4), conv(im4))


# speedup vs baseline: 1.7774x; 1.0594x over previous
import functools

import jax
import jax.numpy as jnp
from jax import lax
from jax.experimental import pallas as pl
from jax.experimental.pallas import tpu as pltpu
from jax.experimental.pallas import tpu_sc as plsc

_B = 4096
_L = 200
_DIM = 16
_N = _B * _L
_V = 1000000

_NC = 2
_NS = 16
_NW = _NC * _NS
_BB = _B // _NW
_QROWS = _V // 4

_FW = 1920
_FCHUNKS = _V // _FW
_FTAILV = _V - _FCHUNKS * _FW
_FPT = (_FCHUNKS + _NW - 1) // _NW


def _build_format():
    mesh = plsc.VectorSubcoreMesh(core_axis_name="c", subcore_axis_name="s")

    @functools.partial(
        pl.kernel,
        mesh=mesh,
        out_type=jax.ShapeDtypeStruct((_QROWS, 128), jnp.float32),
        compiler_params=pltpu.CompilerParams(
            use_tc_tiling_on_sc=True, needs_layout_passes=False
        ),
        scratch_types=[
            pltpu.VMEM((32, _FW), jnp.float32),
            pltpu.VMEM((_FW // 4, 128), jnp.float32),
            pltpu.SemaphoreType.DMA,
            pltpu.SemaphoreType.DMA,
        ],
    )
    def format_table(tbl_t, tail_in, quads_out, colbuf, rowbuf, isem, osem):
        wid = lax.axis_index("s") * _NC + lax.axis_index("c")
        lane = lax.iota(jnp.int32, 16)

        @pl.when(wid == 0)
        def _():
            pltpu.sync_copy(
                tail_in, quads_out.at[pl.ds(_FCHUNKS * _FW // 4, _FTAILV // 4)]
            )

        def do_chunk(c, width):
            v0 = pl.multiple_of(c * _FW, 128)
            pltpu.async_copy(
                tbl_t.at[:, pl.ds(v0, width)],
                colbuf.at[:, pl.ds(0, width)],
                isem,
            ).wait()

            def cbody(cc, carry):
                rot = (lane + cc) % 16

                def mbody(m, carry2):
                    uv = m * 16 + rot
                    rv = uv // 4
                    cbase = (uv % 4) * 32
                    for half in range(2):
                        kv = lane + half * 16
                        v = plsc.load_gather(colbuf, [kv, uv])
                        plsc.store_scatter(rowbuf, [rv, cbase + kv], v)
                    return carry2

                lax.fori_loop(0, width // 16, mbody, 0)
                return carry

            lax.fori_loop(0, 16, cbody, 0)

            pltpu.async_copy(
                rowbuf.at[pl.ds(0, width // 4)],
                quads_out.at[pl.ds(pl.multiple_of(v0 // 4, 8), width // 4)],
                osem,
            ).wait()

        def step(t, carry):
            c = wid + t * _NW

            @pl.when(c < _FCHUNKS)
            def _():
                do_chunk(c, _FW)

            return carry

        lax.fori_loop(0, _FPT, step, 0)

    return format_table


def _build():
    mesh = plsc.VectorSubcoreMesh(core_axis_name="c", subcore_axis_name="s")

    @functools.partial(
        pl.kernel,
        mesh=mesh,
        out_type=(
            jax.ShapeDtypeStruct((_L, 2, _NW, 8, _BB), jnp.float32),
            jax.ShapeDtypeStruct((_L, 2, _NW, 8, _BB), jnp.float32),
        ),
        compiler_params=pltpu.CompilerParams(
            use_tc_tiling_on_sc=True, needs_layout_passes=False
        ),
        scratch_types=[
            pltpu.VMEM((_L, _BB), jnp.int32),
            pltpu.VMEM((_L, _BB), jnp.int32),
            pltpu.VMEM((_BB, _BB), jnp.float32),
            pltpu.VMEM((_BB, _BB), jnp.float32),
            pltpu.VMEM((2, 8, _BB), jnp.float32),
            pltpu.VMEM((2, 8, _BB), jnp.float32),
            pltpu.VMEM((2, 8, _BB), jnp.float32),
            pltpu.VMEM((2, 8, _BB), jnp.float32),
            pltpu.SemaphoreType.DMA,
            pltpu.SemaphoreType.DMA,
            pltpu.SemaphoreType.DMA,
            pltpu.SemaphoreType.DMA,
        ],
    )
    def gather_planes(
        qidx_hbm, off_hbm, quad_hbm, re_hbm, im_hbm,
        qidx_v, off_v, g0, g1, rt0, it0, rt1, it1,
        gsem0, gsem1, osem0, osem1,
    ):
        wid = lax.axis_index("s") * _NC + lax.axis_index("c")

        rows = pl.ds(wid * _L, _L)
        pltpu.sync_copy(qidx_hbm.at[rows], qidx_v)
        pltpu.sync_copy(off_hbm.at[rows], off_v)

        lane = lax.iota(jnp.int32, 16)
        jvs = [m * 16 + lane for m in range(_BB // 16)]

        def fire_gather(l, g, gsem):
            pltpu.async_copy(quad_hbm.at[qidx_v.at[l]], g, gsem)

        def wait_gather(l, g, gsem):
            pltpu.make_async_copy(quad_hbm.at[qidx_v.at[l]], g, gsem).wait()

        def transpose(l, src, dre, dim):
            lv = jnp.full((16,), l, jnp.int32)
            offs = [
                plsc.load_gather(off_v, [lv, jvs[m]])
                for m in range(_BB // 16)
            ]

            def cbody(c, carry):
                rot = (lane + c) % 16
                t1 = rot // 8
                e0 = rot % 8
                for m in range(_BB // 16):
                    jv = jvs[m]
                    col = offs[m] + rot
                    plsc.store_scatter(
                        dre, [t1, e0, jv], plsc.load_gather(src, [jv, col])
                    )
                    plsc.store_scatter(
                        dim, [t1, e0, jv],
                        plsc.load_gather(src, [jv, col + _DIM]),
                    )
                return carry

            lax.fori_loop(0, _DIM, cbody, 0)

        def fire_out(l, rt, it, osem):
            pltpu.async_copy(rt, re_hbm.at[l, :, wid], osem)
            pltpu.async_copy(it, im_hbm.at[l, :, wid], osem)

        def wait_out(l, rt, it, osem):
            pltpu.make_async_copy(rt, re_hbm.at[l, :, wid], osem).wait()
            pltpu.make_async_copy(it, im_hbm.at[l, :, wid], osem).wait()

        fire_gather(0, g0, gsem0)

        def body(i, carry):
            l0 = 2 * i
            l1 = l0 + 1
            fire_gather(l1, g1, gsem1)

            wait_gather(l0, g0, gsem0)

            @pl.when(i > 0)
            def _():
                wait_out(l0 - 2, rt0, it0, osem0)

            transpose(l0, g0, rt0, it0)
            fire_out(l0, rt0, it0, osem0)

            @pl.when(i < _L // 2 - 1)
            def _():
                fire_gather(l0 + 2, g0, gsem0)

            wait_gather(l1, g1, gsem1)

            @pl.when(i > 0)
            def _():
                wait_out(l1 - 2, rt1, it1, osem1)

            transpose(l1, g1, rt1, it1)
            fire_out(l1, rt1, it1, osem1)
            return carry

        lax.fori_loop(0, _L // 2, body, 0)

        wait_out(_L - 2, rt0, it0, osem0)
        wait_out(_L - 1, rt1, it1, osem1)

    return gather_planes


_FORMAT = _build_format()
_GATHER = _build()


def kernel(ids, embed_weight):
    blocked = ids.reshape(_NW, _BB, _L).transpose(0, 2, 1)
    qidx = (blocked // 4).reshape(_NW * _L, _BB)
    off = ((blocked % 4) * 32).reshape(_NW * _L, _BB)
    tail = embed_weight[_FCHUNKS * _FW:].reshape(_FTAILV // 4, 128)
    quads = _FORMAT(embed_weight.T, tail)
    re4, im4 = _GATHER(qidx, off, quads)

    def conv(x):
        return x.transpose(2, 4, 0, 1, 3).reshape(_B, _L, _DIM)

    return lax.complex(conv(re4), conv(im4))


# speedup vs baseline: 1.8165x; 1.0220x over previous
import functools

import jax
import jax.numpy as jnp
from jax import lax
from jax.experimental import pallas as pl
from jax.experimental.pallas import tpu as pltpu
from jax.experimental.pallas import tpu_sc as plsc

_B = 4096
_L = 200
_DIM = 16
_N = _B * _L
_V = 1000000

_NC = 2
_NS = 16
_NW = _NC * _NS
_BB = _B // _NW
_QROWS = _V // 4

_FW = 1920
_FCHUNKS = _V // _FW
_FTAILV = _V - _FCHUNKS * _FW
_FPT = (_FCHUNKS + _NW - 1) // _NW


def _build_format():
    mesh = plsc.VectorSubcoreMesh(core_axis_name="c", subcore_axis_name="s")

    @functools.partial(
        pl.kernel,
        mesh=mesh,
        out_type=jax.ShapeDtypeStruct((_QROWS, 128), jnp.float32),
        compiler_params=pltpu.CompilerParams(
            use_tc_tiling_on_sc=True, needs_layout_passes=False
        ),
        scratch_types=[
            pltpu.VMEM((32, _FW), jnp.float32),
            pltpu.VMEM((_FW // 4, 128), jnp.float32),
            pltpu.SemaphoreType.DMA,
            pltpu.SemaphoreType.DMA,
        ],
    )
    def format_table(tbl_t, tail_in, quads_out, colbuf, rowbuf, isem, osem):
        wid = lax.axis_index("s") * _NC + lax.axis_index("c")
        lane = lax.iota(jnp.int32, 16)

        @pl.when(wid == 0)
        def _():
            pltpu.sync_copy(
                tail_in, quads_out.at[pl.ds(_FCHUNKS * _FW // 4, _FTAILV // 4)]
            )

        def do_chunk(c, width):
            v0 = pl.multiple_of(c * _FW, 128)
            pltpu.async_copy(
                tbl_t.at[:, pl.ds(v0, width)],
                colbuf.at[:, pl.ds(0, width)],
                isem,
            ).wait()

            def cbody(cc, carry):
                rot = (lane + cc) % 16

                def mbody(mg, carry2):
                    for sub in range(4):
                        uv = (mg * 4 + sub) * 16 + rot
                        rv = uv // 4
                        cbase = (uv % 4) * 32
                        for half in range(2):
                            kv = lane + half * 16
                            v = plsc.load_gather(colbuf, [kv, uv])
                            plsc.store_scatter(rowbuf, [rv, cbase + kv], v)
                    return carry2

                lax.fori_loop(0, width // 64, mbody, 0)
                return carry

            lax.fori_loop(0, 16, cbody, 0)

            pltpu.async_copy(
                rowbuf.at[pl.ds(0, width // 4)],
                quads_out.at[pl.ds(pl.multiple_of(v0 // 4, 8), width // 4)],
                osem,
            ).wait()

        def step(t, carry):
            c = wid + t * _NW

            @pl.when(c < _FCHUNKS)
            def _():
                do_chunk(c, _FW)

            return carry

        lax.fori_loop(0, _FPT, step, 0)

    return format_table


def _build():
    mesh = plsc.VectorSubcoreMesh(core_axis_name="c", subcore_axis_name="s")

    @functools.partial(
        pl.kernel,
        mesh=mesh,
        out_type=(
            jax.ShapeDtypeStruct((_L, 2, _NW, 8, _BB), jnp.float32),
            jax.ShapeDtypeStruct((_L, 2, _NW, 8, _BB), jnp.float32),
        ),
        compiler_params=pltpu.CompilerParams(
            use_tc_tiling_on_sc=True, needs_layout_passes=False
        ),
        scratch_types=[
            pltpu.VMEM((_L, _BB), jnp.int32),
            pltpu.VMEM((_L, _BB), jnp.int32),
            pltpu.VMEM((_BB, _BB), jnp.float32),
            pltpu.VMEM((_BB, _BB), jnp.float32),
            pltpu.VMEM((2, 8, _BB), jnp.float32),
            pltpu.VMEM((2, 8, _BB), jnp.float32),
            pltpu.VMEM((2, 8, _BB), jnp.float32),
            pltpu.VMEM((2, 8, _BB), jnp.float32),
            pltpu.SemaphoreType.DMA,
            pltpu.SemaphoreType.DMA,
            pltpu.SemaphoreType.DMA,
            pltpu.SemaphoreType.DMA,
        ],
    )
    def gather_planes(
        qidx_hbm, off_hbm, quad_hbm, re_hbm, im_hbm,
        qidx_v, off_v, g0, g1, rt0, it0, rt1, it1,
        gsem0, gsem1, osem0, osem1,
    ):
        wid = lax.axis_index("s") * _NC + lax.axis_index("c")

        rows = pl.ds(wid * _L, _L)
        pltpu.sync_copy(qidx_hbm.at[rows], qidx_v)
        pltpu.sync_copy(off_hbm.at[rows], off_v)

        lane = lax.iota(jnp.int32, 16)
        jvs = [m * 16 + lane for m in range(_BB // 16)]

        def fire_gather(l, g, gsem):
            pltpu.async_copy(quad_hbm.at[qidx_v.at[l]], g, gsem)

        def wait_gather(l, g, gsem):
            pltpu.make_async_copy(quad_hbm.at[qidx_v.at[l]], g, gsem).wait()

        def transpose(l, src, dre, dim):
            lv = jnp.full((16,), l, jnp.int32)
            offs = [
                plsc.load_gather(off_v, [lv, jvs[m]])
                for m in range(_BB // 16)
            ]

            def cbody(c, carry):
                rot = (lane + c) % 16
                t1 = rot // 8
                e0 = rot % 8
                for m in range(_BB // 16):
                    jv = jvs[m]
                    col = offs[m] + rot
                    plsc.store_scatter(
                        dre, [t1, e0, jv], plsc.load_gather(src, [jv, col])
                    )
                    plsc.store_scatter(
                        dim, [t1, e0, jv],
                        plsc.load_gather(src, [jv, col + _DIM]),
                    )
                return carry

            lax.fori_loop(0, _DIM, cbody, 0)

        def fire_out(l, rt, it, osem):
            pltpu.async_copy(rt, re_hbm.at[l, :, wid], osem)
            pltpu.async_copy(it, im_hbm.at[l, :, wid], osem)

        def wait_out(l, rt, it, osem):
            pltpu.make_async_copy(rt, re_hbm.at[l, :, wid], osem).wait()
            pltpu.make_async_copy(it, im_hbm.at[l, :, wid], osem).wait()

        fire_gather(0, g0, gsem0)

        def body(i, carry):
            l0 = 2 * i
            l1 = l0 + 1
            fire_gather(l1, g1, gsem1)

            wait_gather(l0, g0, gsem0)

            @pl.when(i > 0)
            def _():
                wait_out(l0 - 2, rt0, it0, osem0)

            transpose(l0, g0, rt0, it0)
            fire_out(l0, rt0, it0, osem0)

            @pl.when(i < _L // 2 - 1)
            def _():
                fire_gather(l0 + 2, g0, gsem0)

            wait_gather(l1, g1, gsem1)

            @pl.when(i > 0)
            def _():
                wait_out(l1 - 2, rt1, it1, osem1)

            transpose(l1, g1, rt1, it1)
            fire_out(l1, rt1, it1, osem1)
            return carry

        lax.fori_loop(0, _L // 2, body, 0)

        wait_out(_L - 2, rt0, it0, osem0)
        wait_out(_L - 1, rt1, it1, osem1)

    return gather_planes


_FORMAT = _build_format()
_GATHER = _build()


def kernel(ids, embed_weight):
    blocked = ids.reshape(_NW, _BB, _L).transpose(0, 2, 1)
    qidx = (blocked // 4).reshape(_NW * _L, _BB)
    off = ((blocked % 4) * 32).reshape(_NW * _L, _BB)
    tail = embed_weight[_FCHUNKS * _FW:].reshape(_FTAILV // 4, 128)
    quads = _FORMAT(embed_weight.T, tail)
    re4, im4 = _GATHER(qidx, off, quads)

    def conv(x):
        return x.transpose(2, 4, 0, 1, 3).reshape(_B, _L, _DIM)

    return lax.complex(conv(re4), conv(im4))


# speedup vs baseline: 1.8195x; 1.0016x over previous
import functools

import jax
import jax.numpy as jnp
from jax import lax
from jax.experimental import pallas as pl
from jax.experimental.pallas import tpu as pltpu
from jax.experimental.pallas import tpu_sc as plsc

_B = 4096
_L = 200
_DIM = 16
_N = _B * _L
_V = 1000000

_NC = 2
_NS = 16
_NW = _NC * _NS
_BB = _B // _NW
_QROWS = _V // 4

_FW = 1920
_FCHUNKS = _V // _FW
_FTAILV = _V - _FCHUNKS * _FW
_FPT = (_FCHUNKS + _NW - 1) // _NW


def _build_format():
    mesh = plsc.VectorSubcoreMesh(core_axis_name="c", subcore_axis_name="s")

    @functools.partial(
        pl.kernel,
        mesh=mesh,
        out_type=jax.ShapeDtypeStruct((_QROWS, 128), jnp.float32),
        compiler_params=pltpu.CompilerParams(
            use_tc_tiling_on_sc=True, needs_layout_passes=False
        ),
        scratch_types=[
            pltpu.VMEM((32, _FW), jnp.float32),
            pltpu.VMEM((_FW // 4, 128), jnp.float32),
            pltpu.SemaphoreType.DMA,
            pltpu.SemaphoreType.DMA,
        ],
    )
    def format_table(tbl_t, tail_in, quads_out, colbuf, rowbuf, isem, osem):
        wid = lax.axis_index("s") * _NC + lax.axis_index("c")
        lane = lax.iota(jnp.int32, 16)

        @pl.when(wid == 0)
        def _():
            pltpu.sync_copy(
                tail_in, quads_out.at[pl.ds(_FCHUNKS * _FW // 4, _FTAILV // 4)]
            )

        def do_chunk(c, width):
            v0 = pl.multiple_of(c * _FW, 128)
            pltpu.async_copy(
                tbl_t.at[:, pl.ds(v0, width)],
                colbuf.at[:, pl.ds(0, width)],
                isem,
            ).wait()

            def cbody(cc, carry):
                rot = (lane + cc) % 16

                def mbody(mg, carry2):
                    for sub in range(8):
                        uv = (mg * 8 + sub) * 16 + rot
                        rv = uv // 4
                        cbase = (uv % 4) * 32
                        for half in range(2):
                            kv = lane + half * 16
                            v = plsc.load_gather(colbuf, [kv, uv])
                            plsc.store_scatter(rowbuf, [rv, cbase + kv], v)
                    return carry2

                lax.fori_loop(0, width // 128, mbody, 0)
                return carry

            lax.fori_loop(0, 16, cbody, 0)

            pltpu.async_copy(
                rowbuf.at[pl.ds(0, width // 4)],
                quads_out.at[pl.ds(pl.multiple_of(v0 // 4, 8), width // 4)],
                osem,
            ).wait()

        def step(t, carry):
            c = wid + t * _NW

            @pl.when(c < _FCHUNKS)
            def _():
                do_chunk(c, _FW)

            return carry

        lax.fori_loop(0, _FPT, step, 0)

    return format_table


def _build():
    mesh = plsc.VectorSubcoreMesh(core_axis_name="c", subcore_axis_name="s")

    @functools.partial(
        pl.kernel,
        mesh=mesh,
        out_type=(
            jax.ShapeDtypeStruct((_L, 2, _NW, 8, _BB), jnp.float32),
            jax.ShapeDtypeStruct((_L, 2, _NW, 8, _BB), jnp.float32),
        ),
        compiler_params=pltpu.CompilerParams(
            use_tc_tiling_on_sc=True, needs_layout_passes=False
        ),
        scratch_types=[
            pltpu.VMEM((_L, _BB), jnp.int32),
            pltpu.VMEM((_L, _BB), jnp.int32),
            pltpu.VMEM((_BB, _BB), jnp.float32),
            pltpu.VMEM((_BB, _BB), jnp.float32),
            pltpu.VMEM((2, 8, _BB), jnp.float32),
            pltpu.VMEM((2, 8, _BB), jnp.float32),
            pltpu.VMEM((2, 8, _BB), jnp.float32),
            pltpu.VMEM((2, 8, _BB), jnp.float32),
            pltpu.SemaphoreType.DMA,
            pltpu.SemaphoreType.DMA,
            pltpu.SemaphoreType.DMA,
            pltpu.SemaphoreType.DMA,
        ],
    )
    def gather_planes(
        qidx_hbm, off_hbm, quad_hbm, re_hbm, im_hbm,
        qidx_v, off_v, g0, g1, rt0, it0, rt1, it1,
        gsem0, gsem1, osem0, osem1,
    ):
        wid = lax.axis_index("s") * _NC + lax.axis_index("c")

        rows = pl.ds(wid * _L, _L)
        pltpu.sync_copy(qidx_hbm.at[rows], qidx_v)
        pltpu.sync_copy(off_hbm.at[rows], off_v)

        lane = lax.iota(jnp.int32, 16)
        jvs = [m * 16 + lane for m in range(_BB // 16)]

        def fire_gather(l, g, gsem):
            pltpu.async_copy(quad_hbm.at[qidx_v.at[l]], g, gsem)

        def wait_gather(l, g, gsem):
            pltpu.make_async_copy(quad_hbm.at[qidx_v.at[l]], g, gsem).wait()

        def transpose(l, src, dre, dim):
            lv = jnp.full((16,), l, jnp.int32)
            offs = [
                plsc.load_gather(off_v, [lv, jvs[m]])
                for m in range(_BB // 16)
            ]

            def cbody(c, carry):
                rot = (lane + c) % 16
                t1 = rot // 8
                e0 = rot % 8
                for m in range(_BB // 16):
                    jv = jvs[m]
                    col = offs[m] + rot
                    plsc.store_scatter(
                        dre, [t1, e0, jv], plsc.load_gather(src, [jv, col])
                    )
                    plsc.store_scatter(
                        dim, [t1, e0, jv],
                        plsc.load_gather(src, [jv, col + _DIM]),
                    )
                return carry

            lax.fori_loop(0, _DIM, cbody, 0)

        def fire_out(l, rt, it, osem):
            pltpu.async_copy(rt, re_hbm.at[l, :, wid], osem)
            pltpu.async_copy(it, im_hbm.at[l, :, wid], osem)

        def wait_out(l, rt, it, osem):
            pltpu.make_async_copy(rt, re_hbm.at[l, :, wid], osem).wait()
            pltpu.make_async_copy(it, im_hbm.at[l, :, wid], osem).wait()

        fire_gather(0, g0, gsem0)

        def body(i, carry):
            l0 = 2 * i
            l1 = l0 + 1
            fire_gather(l1, g1, gsem1)

            wait_gather(l0, g0, gsem0)

            @pl.when(i > 0)
            def _():
                wait_out(l0 - 2, rt0, it0, osem0)

            transpose(l0, g0, rt0, it0)
            fire_out(l0, rt0, it0, osem0)

            @pl.when(i < _L // 2 - 1)
            def _():
                fire_gather(l0 + 2, g0, gsem0)

            wait_gather(l1, g1, gsem1)

            @pl.when(i > 0)
            def _():
                wait_out(l1 - 2, rt1, it1, osem1)

            transpose(l1, g1, rt1, it1)
            fire_out(l1, rt1, it1, osem1)
            return carry

        lax.fori_loop(0, _L // 2, body, 0)

        wait_out(_L - 2, rt0, it0, osem0)
        wait_out(_L - 1, rt1, it1, osem1)

    return gather_planes


_FORMAT = _build_format()
_GATHER = _build()


def kernel(ids, embed_weight):
    blocked = ids.reshape(_NW, _BB, _L).transpose(0, 2, 1)
    qidx = (blocked // 4).reshape(_NW * _L, _BB)
    off = ((blocked % 4) * 32).reshape(_NW * _L, _BB)
    tail = embed_weight[_FCHUNKS * _FW:].reshape(_FTAILV // 4, 128)
    quads = _FORMAT(embed_weight.T, tail)
    re4, im4 = _GATHER(qidx, off, quads)

    def conv(x):
        return x.transpose(2, 4, 0, 1, 3).reshape(_B, _L, _DIM)

    return lax.complex(conv(re4), conv(im4))


# speedup vs baseline: 1.9617x; 1.0782x over previous
import functools

import jax
import jax.numpy as jnp
from jax import lax
from jax.experimental import pallas as pl
from jax.experimental.pallas import tpu as pltpu
from jax.experimental.pallas import tpu_sc as plsc

_B = 4096
_L = 200
_DIM = 16
_N = _B * _L
_V = 1000000

_NC = 2
_NS = 16
_NW = _NC * _NS
_BB = _B // _NW
_QROWS = _V // 4

_FW = 896
_FCHUNKS = _V // _FW
_FTAILV = _V - _FCHUNKS * _FW
_FPT = (_FCHUNKS + _NW - 1) // _NW


def _build_format():
    mesh = plsc.VectorSubcoreMesh(core_axis_name="c", subcore_axis_name="s")

    @functools.partial(
        pl.kernel,
        mesh=mesh,
        out_type=jax.ShapeDtypeStruct((_QROWS, 128), jnp.float32),
        compiler_params=pltpu.CompilerParams(
            use_tc_tiling_on_sc=True, needs_layout_passes=False
        ),
        scratch_types=[
            pltpu.VMEM((32, _FW), jnp.float32),
            pltpu.VMEM((32, _FW), jnp.float32),
            pltpu.VMEM((_FW // 4, 128), jnp.float32),
            pltpu.VMEM((_FW // 4, 128), jnp.float32),
            pltpu.SemaphoreType.DMA,
            pltpu.SemaphoreType.DMA,
            pltpu.SemaphoreType.DMA,
            pltpu.SemaphoreType.DMA,
        ],
    )
    def format_table(
        tbl_t, tail_in, quads_out,
        colA, colB, rowA, rowB, isemA, isemB, osemA, osemB,
    ):
        wid = lax.axis_index("s") * _NC + lax.axis_index("c")
        lane = lax.iota(jnp.int32, 16)

        @pl.when(wid == 0)
        def _():
            pltpu.sync_copy(
                tail_in, quads_out.at[pl.ds(_FCHUNKS * _FW // 4, _FTAILV // 4)]
            )

        def cix(t):
            return wid + t * _NW

        def in_copy(t, col, isem):
            v0 = pl.multiple_of(cix(t) * _FW, 128)
            return pltpu.make_async_copy(tbl_t.at[:, pl.ds(v0, _FW)], col, isem)

        def out_copy(t, row, osem):
            r0 = pl.multiple_of(cix(t) * (_FW // 4), 8)
            return pltpu.make_async_copy(
                row, quads_out.at[pl.ds(r0, _FW // 4)], osem
            )

        def transpose(col, row):
            def cbody(cc, carry):
                rot = (lane + cc) % 16

                def mbody(mg, carry2):
                    for sub in range(8):
                        uv = (mg * 8 + sub) * 16 + rot
                        rv = uv // 4
                        cbase = (uv % 4) * 32
                        for half in range(2):
                            kv = lane + half * 16
                            v = plsc.load_gather(col, [kv, uv])
                            plsc.store_scatter(row, [rv, cbase + kv], v)
                    return carry2

                lax.fori_loop(0, _FW // 128, mbody, 0)
                return carry

            lax.fori_loop(0, 16, cbody, 0)

        @pl.when(cix(0) < _FCHUNKS)
        def _():
            in_copy(0, colA, isemA).start()

        def body(i, carry):
            t0 = 2 * i
            t1 = t0 + 1

            @pl.when(cix(t1) < _FCHUNKS)
            def _():
                in_copy(t1, colB, isemB).start()

            @pl.when(cix(t0) < _FCHUNKS)
            def _():
                in_copy(t0, colA, isemA).wait()

                @pl.when(i > 0)
                def _():
                    out_copy(t0 - 2, rowA, osemA).wait()

                transpose(colA, rowA)
                out_copy(t0, rowA, osemA).start()

            @pl.when(cix(t0 + 2) < _FCHUNKS)
            def _():
                in_copy(t0 + 2, colA, isemA).start()

            @pl.when(cix(t1) < _FCHUNKS)
            def _():
                in_copy(t1, colB, isemB).wait()

                @pl.when(i > 0)
                def _():
                    out_copy(t1 - 2, rowB, osemB).wait()

                transpose(colB, rowB)
                out_copy(t1, rowB, osemB).start()

            return carry

        lax.fori_loop(0, (_FPT + 1) // 2, body, 0)

        last_even = ((_FPT + 1) // 2) * 2 - 2

        @pl.when(cix(last_even) < _FCHUNKS)
        def _():
            out_copy(last_even, rowA, osemA).wait()

        @pl.when(cix(last_even) >= _FCHUNKS)
        def _():
            out_copy(last_even - 2, rowA, osemA).wait()

        out_copy(last_even - 1, rowB, osemB).wait()

    return format_table


def _build():
    mesh = plsc.VectorSubcoreMesh(core_axis_name="c", subcore_axis_name="s")

    @functools.partial(
        pl.kernel,
        mesh=mesh,
        out_type=(
            jax.ShapeDtypeStruct((_L, 2, _NW, 8, _BB), jnp.float32),
            jax.ShapeDtypeStruct((_L, 2, _NW, 8, _BB), jnp.float32),
        ),
        compiler_params=pltpu.CompilerParams(
            use_tc_tiling_on_sc=True, needs_layout_passes=False
        ),
        scratch_types=[
            pltpu.VMEM((_L, _BB), jnp.int32),
            pltpu.VMEM((_L, _BB), jnp.int32),
            pltpu.VMEM((_BB, _BB), jnp.float32),
            pltpu.VMEM((_BB, _BB), jnp.float32),
            pltpu.VMEM((2, 8, _BB), jnp.float32),
            pltpu.VMEM((2, 8, _BB), jnp.float32),
            pltpu.VMEM((2, 8, _BB), jnp.float32),
            pltpu.VMEM((2, 8, _BB), jnp.float32),
            pltpu.SemaphoreType.DMA,
            pltpu.SemaphoreType.DMA,
            pltpu.SemaphoreType.DMA,
            pltpu.SemaphoreType.DMA,
        ],
    )
    def gather_planes(
        qidx_hbm, off_hbm, quad_hbm, re_hbm, im_hbm,
        qidx_v, off_v, g0, g1, rt0, it0, rt1, it1,
        gsem0, gsem1, osem0, osem1,
    ):
        wid = lax.axis_index("s") * _NC + lax.axis_index("c")

        rows = pl.ds(wid * _L, _L)
        pltpu.sync_copy(qidx_hbm.at[rows], qidx_v)
        pltpu.sync_copy(off_hbm.at[rows], off_v)

        lane = lax.iota(jnp.int32, 16)
        jvs = [m * 16 + lane for m in range(_BB // 16)]

        def fire_gather(l, g, gsem):
            pltpu.async_copy(quad_hbm.at[qidx_v.at[l]], g, gsem)

        def wait_gather(l, g, gsem):
            pltpu.make_async_copy(quad_hbm.at[qidx_v.at[l]], g, gsem).wait()

        def transpose(l, src, dre, dim):
            lv = jnp.full((16,), l, jnp.int32)
            offs = [
                plsc.load_gather(off_v, [lv, jvs[m]])
                for m in range(_BB // 16)
            ]

            def cbody(c, carry):
                rot = (lane + c) % 16
                t1 = rot // 8
                e0 = rot % 8
                for m in range(_BB // 16):
                    jv = jvs[m]
                    col = offs[m] + rot
                    plsc.store_scatter(
                        dre, [t1, e0, jv], plsc.load_gather(src, [jv, col])
                    )
                    plsc.store_scatter(
                        dim, [t1, e0, jv],
                        plsc.load_gather(src, [jv, col + _DIM]),
                    )
                return carry

            lax.fori_loop(0, _DIM, cbody, 0)

        def fire_out(l, rt, it, osem):
            pltpu.async_copy(rt, re_hbm.at[l, :, wid], osem)
            pltpu.async_copy(it, im_hbm.at[l, :, wid], osem)

        def wait_out(l, rt, it, osem):
            pltpu.make_async_copy(rt, re_hbm.at[l, :, wid], osem).wait()
            pltpu.make_async_copy(it, im_hbm.at[l, :, wid], osem).wait()

        fire_gather(0, g0, gsem0)

        def body(i, carry):
            l0 = 2 * i
            l1 = l0 + 1
            fire_gather(l1, g1, gsem1)

            wait_gather(l0, g0, gsem0)

            @pl.when(i > 0)
            def _():
                wait_out(l0 - 2, rt0, it0, osem0)

            transpose(l0, g0, rt0, it0)
            fire_out(l0, rt0, it0, osem0)

            @pl.when(i < _L // 2 - 1)
            def _():
                fire_gather(l0 + 2, g0, gsem0)

            wait_gather(l1, g1, gsem1)

            @pl.when(i > 0)
            def _():
                wait_out(l1 - 2, rt1, it1, osem1)

            transpose(l1, g1, rt1, it1)
            fire_out(l1, rt1, it1, osem1)
            return carry

        lax.fori_loop(0, _L // 2, body, 0)

        wait_out(_L - 2, rt0, it0, osem0)
        wait_out(_L - 1, rt1, it1, osem1)

    return gather_planes


_FORMAT = _build_format()
_GATHER = _build()


def kernel(ids, embed_weight):
    blocked = ids.reshape(_NW, _BB, _L).transpose(0, 2, 1)
    qidx = (blocked // 4).reshape(_NW * _L, _BB)
    off = ((blocked % 4) * 32).reshape(_NW * _L, _BB)
    tail = embed_weight[_FCHUNKS * _FW:].reshape(_FTAILV // 4, 128)
    quads = _FORMAT(embed_weight.T, tail)
    re4, im4 = _GATHER(qidx, off, quads)

    def conv(x):
        return x.transpose(2, 4, 0, 1, 3).reshape(_B, _L, _DIM)

    return lax.complex(conv(re4), conv(im4))


# speedup vs baseline: 1.9639x; 1.0011x over previous
import functools

import jax
import jax.numpy as jnp
from jax import lax
from jax.experimental import pallas as pl
from jax.experimental.pallas import tpu as pltpu
from jax.experimental.pallas import tpu_sc as plsc

_B = 4096
_L = 200
_DIM = 16
_N = _B * _L
_V = 1000000

_NC = 2
_NS = 16
_NW = _NC * _NS
_BB = _B // _NW
_QROWS = _V // 4

_FW = 896
_FCHUNKS = _V // _FW
_FTAILV = _V - _FCHUNKS * _FW
_FPT = (_FCHUNKS + _NW - 1) // _NW


def _build_format():
    mesh = plsc.VectorSubcoreMesh(core_axis_name="c", subcore_axis_name="s")

    @functools.partial(
        pl.kernel,
        mesh=mesh,
        out_type=jax.ShapeDtypeStruct((_QROWS, 128), jnp.float32),
        compiler_params=pltpu.CompilerParams(
            use_tc_tiling_on_sc=True, needs_layout_passes=False
        ),
        scratch_types=[
            pltpu.VMEM((32, _FW), jnp.float32),
            pltpu.VMEM((32, _FW), jnp.float32),
            pltpu.VMEM((_FW // 4, 128), jnp.float32),
            pltpu.VMEM((_FW // 4, 128), jnp.float32),
            pltpu.SemaphoreType.DMA,
            pltpu.SemaphoreType.DMA,
            pltpu.SemaphoreType.DMA,
            pltpu.SemaphoreType.DMA,
        ],
    )
    def format_table(
        tbl_t, tail_in, quads_out,
        colA, colB, rowA, rowB, isemA, isemB, osemA, osemB,
    ):
        wid = lax.axis_index("s") * _NC + lax.axis_index("c")
        lane = lax.iota(jnp.int32, 16)

        @pl.when(wid == 0)
        def _():
            pltpu.sync_copy(
                tail_in, quads_out.at[pl.ds(_FCHUNKS * _FW // 4, _FTAILV // 4)]
            )

        def cix(t):
            return wid + t * _NW

        def in_copy(t, col, isem):
            v0 = pl.multiple_of(cix(t) * _FW, 128)
            return pltpu.make_async_copy(tbl_t.at[:, pl.ds(v0, _FW)], col, isem)

        def out_copy(t, row, osem):
            r0 = pl.multiple_of(cix(t) * (_FW // 4), 8)
            return pltpu.make_async_copy(
                row, quads_out.at[pl.ds(r0, _FW // 4)], osem
            )

        def transpose(col, row):
            def cbody(cc, carry):
                rot = (lane + cc) % 16

                def mbody(mg, carry2):
                    for sub in range(8):
                        uv = (mg * 8 + sub) * 16 + rot
                        rv = uv // 4
                        cbase = (uv % 4) * 32
                        for half in range(2):
                            kv = lane + half * 16
                            v = plsc.load_gather(col, [kv, uv])
                            plsc.store_scatter(row, [rv, cbase + kv], v)
                    return carry2

                lax.fori_loop(0, _FW // 128, mbody, 0)
                return carry

            lax.fori_loop(0, 16, cbody, 0)

        @pl.when(cix(0) < _FCHUNKS)
        def _():
            in_copy(0, colA, isemA).start()

        def body(i, carry):
            t0 = 2 * i
            t1 = t0 + 1

            @pl.when(cix(t1) < _FCHUNKS)
            def _():
                in_copy(t1, colB, isemB).start()

            @pl.when(cix(t0) < _FCHUNKS)
            def _():
                in_copy(t0, colA, isemA).wait()

                @pl.when(i > 0)
                def _():
                    out_copy(t0 - 2, rowA, osemA).wait()

                transpose(colA, rowA)
                out_copy(t0, rowA, osemA).start()

            @pl.when(cix(t0 + 2) < _FCHUNKS)
            def _():
                in_copy(t0 + 2, colA, isemA).start()

            @pl.when(cix(t1) < _FCHUNKS)
            def _():
                in_copy(t1, colB, isemB).wait()

                @pl.when(i > 0)
                def _():
                    out_copy(t1 - 2, rowB, osemB).wait()

                transpose(colB, rowB)
                out_copy(t1, rowB, osemB).start()

            return carry

        lax.fori_loop(0, (_FPT + 1) // 2, body, 0)

        last_even = ((_FPT + 1) // 2) * 2 - 2

        @pl.when(cix(last_even) < _FCHUNKS)
        def _():
            out_copy(last_even, rowA, osemA).wait()

        @pl.when(cix(last_even) >= _FCHUNKS)
        def _():
            out_copy(last_even - 2, rowA, osemA).wait()

        out_copy(last_even - 1, rowB, osemB).wait()

    return format_table


def _build():
    mesh = plsc.VectorSubcoreMesh(core_axis_name="c", subcore_axis_name="s")

    @functools.partial(
        pl.kernel,
        mesh=mesh,
        out_type=(
            jax.ShapeDtypeStruct((_L, 2, _NW, 8, _BB), jnp.float32),
            jax.ShapeDtypeStruct((_L, 2, _NW, 8, _BB), jnp.float32),
        ),
        compiler_params=pltpu.CompilerParams(
            use_tc_tiling_on_sc=True, needs_layout_passes=False
        ),
        scratch_types=[
            pltpu.VMEM((_L, _BB), jnp.int32),
            pltpu.VMEM((_L, _BB), jnp.int32),
            pltpu.VMEM((_BB, _BB), jnp.float32),
            pltpu.VMEM((_BB, _BB), jnp.float32),
            pltpu.VMEM((2, 8, _BB), jnp.float32),
            pltpu.VMEM((2, 8, _BB), jnp.float32),
            pltpu.VMEM((2, 8, _BB), jnp.float32),
            pltpu.VMEM((2, 8, _BB), jnp.float32),
            pltpu.SemaphoreType.DMA,
            pltpu.SemaphoreType.DMA,
            pltpu.SemaphoreType.DMA,
            pltpu.SemaphoreType.DMA,
        ],
    )
    def gather_planes(
        qidx_hbm, off_hbm, quad_hbm, re_hbm, im_hbm,
        qidx_v, off_v, g0, g1, rt0, it0, rt1, it1,
        gsem0, gsem1, osem0, osem1,
    ):
        wid = lax.axis_index("s") * _NC + lax.axis_index("c")

        rows = pl.ds(wid * _L, _L)
        pltpu.sync_copy(qidx_hbm.at[rows], qidx_v)
        pltpu.sync_copy(off_hbm.at[rows], off_v)

        lane = lax.iota(jnp.int32, 16)
        jvs = [m * 16 + lane for m in range(_BB // 16)]

        def fire_gather(l, g, gsem):
            pltpu.async_copy(quad_hbm.at[qidx_v.at[l]], g, gsem)

        def wait_gather(l, g, gsem):
            pltpu.make_async_copy(quad_hbm.at[qidx_v.at[l]], g, gsem).wait()

        def transpose(l, src, dre, dim):
            lv = jnp.full((16,), l, jnp.int32)
            offs = [
                plsc.load_gather(off_v, [lv, jvs[m]])
                for m in range(_BB // 16)
            ]

            def cbody(cg, carry):
                for sub in range(2):
                    rot = (lane + (cg * 2 + sub)) % 16
                    t1 = rot // 8
                    e0 = rot % 8
                    for m in range(_BB // 16):
                        jv = jvs[m]
                        col = offs[m] + rot
                        plsc.store_scatter(
                            dre, [t1, e0, jv], plsc.load_gather(src, [jv, col])
                        )
                        plsc.store_scatter(
                            dim, [t1, e0, jv],
                            plsc.load_gather(src, [jv, col + _DIM]),
                        )
                return carry

            lax.fori_loop(0, _DIM // 2, cbody, 0)

        def fire_out(l, rt, it, osem):
            pltpu.async_copy(rt, re_hbm.at[l, :, wid], osem)
            pltpu.async_copy(it, im_hbm.at[l, :, wid], osem)

        def wait_out(l, rt, it, osem):
            pltpu.make_async_copy(rt, re_hbm.at[l, :, wid], osem).wait()
            pltpu.make_async_copy(it, im_hbm.at[l, :, wid], osem).wait()

        fire_gather(0, g0, gsem0)

        def body(i, carry):
            l0 = 2 * i
            l1 = l0 + 1
            fire_gather(l1, g1, gsem1)

            wait_gather(l0, g0, gsem0)

            @pl.when(i > 0)
            def _():
                wait_out(l0 - 2, rt0, it0, osem0)

            transpose(l0, g0, rt0, it0)
            fire_out(l0, rt0, it0, osem0)

            @pl.when(i < _L // 2 - 1)
            def _():
                fire_gather(l0 + 2, g0, gsem0)

            wait_gather(l1, g1, gsem1)

            @pl.when(i > 0)
            def _():
                wait_out(l1 - 2, rt1, it1, osem1)

            transpose(l1, g1, rt1, it1)
            fire_out(l1, rt1, it1, osem1)
            return carry

        lax.fori_loop(0, _L // 2, body, 0)

        wait_out(_L - 2, rt0, it0, osem0)
        wait_out(_L - 1, rt1, it1, osem1)

    return gather_planes


_FORMAT = _build_format()
_GATHER = _build()


def kernel(ids, embed_weight):
    blocked = ids.reshape(_NW, _BB, _L).transpose(0, 2, 1)
    qidx = (blocked // 4).reshape(_NW * _L, _BB)
    off = ((blocked % 4) * 32).reshape(_NW * _L, _BB)
    tail = embed_weight[_FCHUNKS * _FW:].reshape(_FTAILV // 4, 128)
    quads = _FORMAT(embed_weight.T, tail)
    re4, im4 = _GATHER(qidx, off, quads)

    def conv(x):
        return x.transpose(2, 4, 0, 1, 3).reshape(_B, _L, _DIM)

    return lax.complex(conv(re4), conv(im4))
